# baseline (speedup 1.0000x reference)
import functools
import math

import jax
import jax.numpy as jnp
from jax import lax
from jax.experimental import pallas as pl
from jax.experimental.pallas import tpu as pltpu
from jax.experimental.pallas import tpu_sc as plsc

_SMOOTH = 0.1


@functools.cache
def _sc_part(N, V, n_sc):
    info = plsc.get_sparse_core_info()
    nc, ns, nl = info.num_cores, info.num_subcores, info.num_lanes
    nw = nc * ns
    per_w = N // nw
    eps = _SMOOTH / (V - 2)
    cconst = eps * (V - 2) * math.log(eps) + (1.0 - _SMOOTH) * math.log(1.0 - _SMOOTH)
    mesh = plsc.VectorSubcoreMesh(core_axis_name="c", subcore_axis_name="s")

    cw = 1408
    vmain = (V // 128) * 128
    nch = vmain // cw
    assert nch * cw == vmain
    tpw = n_sc // 8 // nw
    assert tpw * 8 * nw == n_sc
    n_tc = N - n_sc
    y2len = max(nl, 8 * tpw)

    @functools.partial(
        pl.kernel,
        mesh=mesh,
        out_type=(
            jax.ShapeDtypeStruct((nw * nl,), jnp.float32),
            jax.ShapeDtypeStruct((nw * nl,), jnp.float32),
        ),
        scratch_types=[
            pltpu.SMEM((per_w,), jnp.int32),
            pltpu.VMEM((per_w,), jnp.int32),
            pltpu.VMEM((per_w, 8, 128), jnp.float32),
            pltpu.VMEM((per_w, 128), jnp.float32),
            pltpu.VMEM((nl,), jnp.float32),
            pltpu.SMEM((y2len,), jnp.int32),
            pltpu.VMEM((y2len,), jnp.int32),
            pltpu.VMEM((8, cw), jnp.float32),
            pltpu.VMEM((8, cw), jnp.float32),
            pltpu.VMEM((nl,), jnp.float32),
            pltpu.SemaphoreType.DMA,
            pltpu.SemaphoreType.DMA,
            pltpu.SemaphoreType.DMA,
        ],
    )
    def sc_kernel(
        x2d, yh, outh, out2h, y_s, y_vm, g_v, z_v, c_v, y2_s, y2_vm,
        sbuf, sbuf2, s_v, sem, semA, semB
    ):
        wid = lax.axis_index("s") * nc + lax.axis_index("c")
        base = wid * per_w
        pltpu.sync_copy(yh.at[pl.ds(base, per_w)], y_vm)
        for c in range(per_w // nl):
            yv16 = y_vm[pl.ds(c * nl, nl)]
            for j in range(nl):
                y_s[c * nl + j] = yv16[j]
        pltpu.sync_copy(x2d.at[pl.ds(base, per_w), pl.ds(0, 128)], z_v)
        for r in range(per_w):
            col = pl.multiple_of((y_s[r] // 128) * 128, 128)
            row0 = base + (r // 8) * 8
            pltpu.sync_copy(x2d.at[pl.ds(row0, 8), pl.ds(col, 128)], g_v.at[r])
        lanes = lax.iota(jnp.int32, nl)
        one = jnp.full((nl,), 1, jnp.int32)
        m0 = (one - jnp.minimum(lanes, one)).astype(jnp.float32)
        cvec0 = m0 * cconst
        acc = jnp.zeros((nl,), jnp.float32)
        for r in range(per_w):
            y_r = y_s[r]
            yv = jnp.full((nl,), y_r, jnp.int32)
            lc = pl.multiple_of(((y_r % 128) // nl) * nl, nl)
            gv = g_v[r, r % 8, pl.ds(lc, nl)]
            zv = z_v[r, pl.ds(0, nl)]
            gm = (one - jnp.minimum(jnp.abs(lanes - yv % nl), one)).astype(
                jnp.float32
            )
            pm = jnp.minimum(yv, one).astype(jnp.float32)
            row_vec = (
                gv * gm * (eps - (1.0 - _SMOOTH)) + zv * (m0 * eps) + cvec0
            )
            acc = acc + row_vec * pm
        c_v[...] = acc
        pltpu.sync_copy(c_v, outh.at[pl.ds(wid * nl, nl)])

        if tpw > 0:
            srow_base = n_tc + wid * 8 * tpw
            pltpu.sync_copy(
                yh.at[pl.ds(srow_base, 8 * tpw)], y2_vm.at[pl.ds(0, 8 * tpw)]
            )
            for c in range((8 * tpw + nl - 1) // nl):
                yv16 = y2_vm[pl.ds(c * nl, nl)]
                for j in range(nl):
                    if c * nl + j < 8 * tpw:
                        y2_s[c * nl + j] = yv16[j]
            sacc = jnp.zeros((nl,), jnp.float32)

            def _row_sums(buf, acc_c, pms):
                for r in range(8):

                    def inner(i, ps, r=r):
                        a, b = ps
                        for u in range(4):
                            offa = pl.multiple_of(i * 128 + u * 32, nl)
                            a = a + buf[r, pl.ds(offa, nl)]
                            b = b + buf[r, pl.ds(offa + nl, nl)]
                        return (a, b)

                    pa, pb = lax.fori_loop(
                        0, cw // 128,
                        inner,
                        (jnp.zeros((nl,), jnp.float32), jnp.zeros((nl,), jnp.float32)),
                    )
                    acc_c = acc_c + (pa + pb) * pms[r]
                return acc_c

            for t in range(tpw):
                row0 = pl.multiple_of(srow_base + t * 8, 8)
                pms = []
                for r in range(8):
                    yv = jnp.full((nl,), y2_s[t * 8 + r], jnp.int32)
                    pms.append(jnp.minimum(yv, one).astype(jnp.float32))

                def _issue(c, buf, s):
                    col = pl.multiple_of(c * cw, 128)
                    pltpu.async_copy(
                        x2d.at[pl.ds(row0, 8), pl.ds(col, cw)], buf, s
                    )

                def _wait(buf, s):
                    pltpu.make_async_copy(
                        x2d.at[pl.ds(row0, 8), pl.ds(0, cw)], buf, s
                    ).wait()

                _issue(0, sbuf, semA)
                _issue(1, sbuf2, semB)

                def pair_body(i, acc_c, pms=pms):
                    c = i * 2
                    _wait(sbuf, semA)
                    acc_c = _row_sums(sbuf, acc_c, pms)
                    _issue(c + 2, sbuf, semA)
                    _wait(sbuf2, semB)
                    acc_c = _row_sums(sbuf2, acc_c, pms)

                    @pl.when(c + 3 < nch)
                    def _():
                        _issue(c + 3, sbuf2, semB)

                    return acc_c

                sacc = lax.fori_loop(0, (nch - 1) // 2, pair_body, sacc)
                _wait(sbuf, semA)
                sacc = _row_sums(sbuf, sacc, pms)
            s_v[...] = sacc
            pltpu.sync_copy(s_v, out2h.at[pl.ds(wid * nl, nl)])
        else:
            s_v[...] = jnp.zeros((nl,), jnp.float32)
            pltpu.sync_copy(s_v, out2h.at[pl.ds(wid * nl, nl)])

    return sc_kernel


@functools.cache
def _tc_loss(n_tc, n_all, V, br):
    nblk = n_tc // br
    vmain = (V // 128) * 128
    vtail = V - vmain

    if n_tc == 0:
        def tail_body(xt_ref, yf_ref, out_ref):
            tmask = lax.broadcasted_iota(jnp.int32, (n_all, 128), 1) < vtail
            ts = jnp.sum(
                jnp.where(tmask, xt_ref[...], 0.0), axis=1, keepdims=True
            )
            ts = jnp.where(yf_ref[...] != 0, ts, 0.0)
            out_ref[0, 0] = jnp.sum(ts)

        return pl.pallas_call(
            tail_body,
            grid=(1,),
            in_specs=[
                pl.BlockSpec((n_all, 128), lambda i: (0, vmain // 128)),
                pl.BlockSpec((n_all, 1), lambda i: (0, 0)),
            ],
            out_specs=pl.BlockSpec(
                (1, 1), lambda i: (0, 0), memory_space=pltpu.SMEM
            ),
            out_shape=jax.ShapeDtypeStruct((1, 1), jnp.float32),
        )

    def body(xm_ref, xt_ref, y_ref, yf_ref, out_ref, acc_ref):
        pid = pl.program_id(0)

        @pl.when(pid == 0)
        def _():
            tmask = lax.broadcasted_iota(jnp.int32, (n_all, 128), 1) < vtail
            ts = jnp.sum(
                jnp.where(tmask, xt_ref[...], 0.0), axis=1, keepdims=True
            )
            ts = jnp.where(yf_ref[...] != 0, ts, 0.0)
            acc_ref[0] = jnp.sum(ts)

        srow = jnp.sum(xm_ref[...], axis=1, keepdims=True)
        srow = jnp.where(y_ref[...] != 0, srow, 0.0)
        acc_ref[0] += jnp.sum(srow)

        @pl.when(pid == nblk - 1)
        def _():
            out_ref[0, 0] = acc_ref[0]

    return pl.pallas_call(
        body,
        grid=(nblk,),
        in_specs=[
            pl.BlockSpec((br, vmain), lambda i: (i, 0)),
            pl.BlockSpec((n_all, 128), lambda i: (0, vmain // 128)),
            pl.BlockSpec((br, 1), lambda i: (i, 0)),
            pl.BlockSpec((n_all, 1), lambda i: (0, 0)),
        ],
        out_specs=pl.BlockSpec((1, 1), lambda i: (0, 0), memory_space=pltpu.SMEM),
        out_shape=jax.ShapeDtypeStruct((1, 1), jnp.float32),
        scratch_shapes=[pltpu.SMEM((1,), jnp.float32)],
    )


def kernel(x, y, norm):
    V = x.shape[-1]
    x2 = x.reshape(-1, V)
    N = x2.shape[0]
    yf = y.reshape(-1).astype(jnp.int32)
    n_sc = N
    contrib, ssum = _sc_part(N, V, n_sc)(x2, yf)
    y2 = yf.reshape(N, 1)
    if n_sc == N:
        tc_total = _tc_loss(0, N, V, 64)(x2, y2)[0, 0]
    else:
        tc_total = _tc_loss(N - n_sc, N, V, 64)(x2, x2, y2, y2)[0, 0]
    eps = _SMOOTH / (V - 2)
    masked_total = tc_total + jnp.sum(ssum)
    return (jnp.sum(contrib) - eps * masked_total) / norm


# speedup vs baseline: 1.2376x; 1.2376x over previous
import functools
import math

import jax
import jax.numpy as jnp
from jax import lax
from jax.experimental import pallas as pl
from jax.experimental.pallas import tpu as pltpu
from jax.experimental.pallas import tpu_sc as plsc

_SMOOTH = 0.1


@functools.cache
def _sc_part(N, V, n_sc):
    info = plsc.get_sparse_core_info()
    nc, ns, nl = info.num_cores, info.num_subcores, info.num_lanes
    nw = nc * ns
    per_w = N // nw
    eps = _SMOOTH / (V - 2)
    cconst = eps * (V - 2) * math.log(eps) + (1.0 - _SMOOTH) * math.log(1.0 - _SMOOTH)
    mesh = plsc.VectorSubcoreMesh(core_axis_name="c", subcore_axis_name="s")

    cw = 1408
    vmain = (V // 128) * 128
    nch = vmain // cw
    assert nch * cw == vmain
    tpw = n_sc // 8 // nw
    assert tpw * 8 * nw == n_sc
    n_tc = N - n_sc
    y2len = max(nl, 8 * tpw)

    @functools.partial(
        pl.kernel,
        mesh=mesh,
        out_type=(
            jax.ShapeDtypeStruct((nw * nl,), jnp.float32),
            jax.ShapeDtypeStruct((nw * nl,), jnp.float32),
        ),
        scratch_types=[
            pltpu.SMEM((per_w,), jnp.int32),
            pltpu.VMEM((per_w,), jnp.int32),
            pltpu.VMEM((per_w, 8, 128), jnp.float32),
            pltpu.VMEM((per_w, 128), jnp.float32),
            pltpu.VMEM((nl,), jnp.float32),
            pltpu.SMEM((y2len,), jnp.int32),
            pltpu.VMEM((y2len,), jnp.int32),
            pltpu.VMEM((8, cw), jnp.float32),
            pltpu.VMEM((8, cw), jnp.float32),
            pltpu.VMEM((nl,), jnp.float32),
            pltpu.SemaphoreType.DMA,
            pltpu.SemaphoreType.DMA,
            pltpu.SemaphoreType.DMA,
        ],
    )
    def sc_kernel(
        x2d, yh, outh, out2h, y_s, y_vm, g_v, z_v, c_v, y2_s, y2_vm,
        sbuf, sbuf2, s_v, sem, semA, semB
    ):
        wid = lax.axis_index("s") * nc + lax.axis_index("c")
        base = wid * per_w
        pltpu.sync_copy(yh.at[pl.ds(base, per_w)], y_vm)
        for c in range(per_w // nl):
            yv16 = y_vm[pl.ds(c * nl, nl)]
            for j in range(nl):
                y_s[c * nl + j] = yv16[j]
        pltpu.sync_copy(x2d.at[pl.ds(base, per_w), pl.ds(0, 128)], z_v)
        for r in range(per_w):
            col = pl.multiple_of((y_s[r] // 128) * 128, 128)
            row0 = base + (r // 8) * 8
            pltpu.sync_copy(x2d.at[pl.ds(row0, 8), pl.ds(col, 128)], g_v.at[r])
        lanes = lax.iota(jnp.int32, nl)
        one = jnp.full((nl,), 1, jnp.int32)
        m0 = (one - jnp.minimum(lanes, one)).astype(jnp.float32)
        cvec0 = m0 * cconst
        acc = jnp.zeros((nl,), jnp.float32)
        for r in range(per_w):
            y_r = y_s[r]
            yv = jnp.full((nl,), y_r, jnp.int32)
            lc = pl.multiple_of(((y_r % 128) // nl) * nl, nl)
            gv = g_v[r, r % 8, pl.ds(lc, nl)]
            zv = z_v[r, pl.ds(0, nl)]
            gm = (one - jnp.minimum(jnp.abs(lanes - yv % nl), one)).astype(
                jnp.float32
            )
            pm = jnp.minimum(yv, one).astype(jnp.float32)
            row_vec = (
                gv * gm * (eps - (1.0 - _SMOOTH)) + zv * (m0 * eps) + cvec0
            )
            acc = acc + row_vec * pm
        c_v[...] = acc
        pltpu.sync_copy(c_v, outh.at[pl.ds(wid * nl, nl)])

        if tpw > 0:
            srow_base = n_tc + wid * 8 * tpw
            pltpu.sync_copy(
                yh.at[pl.ds(srow_base, 8 * tpw)], y2_vm.at[pl.ds(0, 8 * tpw)]
            )
            for c in range((8 * tpw + nl - 1) // nl):
                yv16 = y2_vm[pl.ds(c * nl, nl)]
                for j in range(nl):
                    if c * nl + j < 8 * tpw:
                        y2_s[c * nl + j] = yv16[j]
            sacc = jnp.zeros((nl,), jnp.float32)

            def _row_sums(buf, acc_c, pms):
                for r in range(8):

                    def inner(i, ps, r=r):
                        a, b = ps
                        for u in range(4):
                            offa = pl.multiple_of(i * 128 + u * 32, nl)
                            a = a + buf[r, pl.ds(offa, nl)]
                            b = b + buf[r, pl.ds(offa + nl, nl)]
                        return (a, b)

                    pa, pb = lax.fori_loop(
                        0, cw // 128,
                        inner,
                        (jnp.zeros((nl,), jnp.float32), jnp.zeros((nl,), jnp.float32)),
                    )
                    acc_c = acc_c + (pa + pb) * pms[r]
                return acc_c

            for t in range(tpw):
                row0 = pl.multiple_of(srow_base + t * 8, 8)
                pms = []
                for r in range(8):
                    yv = jnp.full((nl,), y2_s[t * 8 + r], jnp.int32)
                    pms.append(jnp.minimum(yv, one).astype(jnp.float32))

                def _issue(c, buf, s):
                    col = pl.multiple_of(c * cw, 128)
                    pltpu.async_copy(
                        x2d.at[pl.ds(row0, 8), pl.ds(col, cw)], buf, s
                    )

                def _wait(buf, s):
                    pltpu.make_async_copy(
                        x2d.at[pl.ds(row0, 8), pl.ds(0, cw)], buf, s
                    ).wait()

                _issue(0, sbuf, semA)
                _issue(1, sbuf2, semB)

                def pair_body(i, acc_c, pms=pms):
                    c = i * 2
                    _wait(sbuf, semA)
                    acc_c = _row_sums(sbuf, acc_c, pms)
                    _issue(c + 2, sbuf, semA)
                    _wait(sbuf2, semB)
                    acc_c = _row_sums(sbuf2, acc_c, pms)

                    @pl.when(c + 3 < nch)
                    def _():
                        _issue(c + 3, sbuf2, semB)

                    return acc_c

                sacc = lax.fori_loop(0, (nch - 1) // 2, pair_body, sacc)
                _wait(sbuf, semA)
                sacc = _row_sums(sbuf, sacc, pms)
            s_v[...] = sacc
            pltpu.sync_copy(s_v, out2h.at[pl.ds(wid * nl, nl)])
        else:
            s_v[...] = jnp.zeros((nl,), jnp.float32)
            pltpu.sync_copy(s_v, out2h.at[pl.ds(wid * nl, nl)])

    return sc_kernel


@functools.cache
def _tc_loss(n_tc, n_all, V, br):
    nblk = n_tc // br
    vmain = (V // 128) * 128
    vtail = V - vmain

    if n_tc == 0:
        def tail_body(xt_ref, yf_ref, out_ref):
            tmask = lax.broadcasted_iota(jnp.int32, (n_all, 128), 1) < vtail
            ts = jnp.sum(
                jnp.where(tmask, xt_ref[...], 0.0), axis=1, keepdims=True
            )
            ts = jnp.where(yf_ref[...] != 0, ts, 0.0)
            out_ref[0, 0] = jnp.sum(ts)

        return pl.pallas_call(
            tail_body,
            grid=(1,),
            in_specs=[
                pl.BlockSpec((n_all, 128), lambda i: (0, vmain // 128)),
                pl.BlockSpec((n_all, 1), lambda i: (0, 0)),
            ],
            out_specs=pl.BlockSpec(
                (1, 1), lambda i: (0, 0), memory_space=pltpu.SMEM
            ),
            out_shape=jax.ShapeDtypeStruct((1, 1), jnp.float32),
        )

    def body(xm_ref, xt_ref, y_ref, yf_ref, out_ref, acc_ref):
        pid = pl.program_id(0)

        @pl.when(pid == 0)
        def _():
            tmask = lax.broadcasted_iota(jnp.int32, (n_all, 128), 1) < vtail
            ts = jnp.sum(
                jnp.where(tmask, xt_ref[...], 0.0), axis=1, keepdims=True
            )
            ts = jnp.where(yf_ref[...] != 0, ts, 0.0)
            acc_ref[0] = jnp.sum(ts)

        srow = jnp.sum(xm_ref[...], axis=1, keepdims=True)
        srow = jnp.where(y_ref[...] != 0, srow, 0.0)
        acc_ref[0] += jnp.sum(srow)

        @pl.when(pid == nblk - 1)
        def _():
            out_ref[0, 0] = acc_ref[0]

    return pl.pallas_call(
        body,
        grid=(nblk,),
        in_specs=[
            pl.BlockSpec((br, vmain), lambda i: (i, 0)),
            pl.BlockSpec((n_all, 128), lambda i: (0, vmain // 128)),
            pl.BlockSpec((br, 1), lambda i: (i, 0)),
            pl.BlockSpec((n_all, 1), lambda i: (0, 0)),
        ],
        out_specs=pl.BlockSpec((1, 1), lambda i: (0, 0), memory_space=pltpu.SMEM),
        out_shape=jax.ShapeDtypeStruct((1, 1), jnp.float32),
        scratch_shapes=[pltpu.SMEM((1,), jnp.float32)],
    )


def kernel(x, y, norm):
    V = x.shape[-1]
    x2 = x.reshape(-1, V)
    N = x2.shape[0]
    yf = y.reshape(-1).astype(jnp.int32)
    n_sc = 0
    contrib, ssum = _sc_part(N, V, n_sc)(x2, yf)
    y2 = yf.reshape(N, 1)
    if n_sc == N:
        tc_total = _tc_loss(0, N, V, 64)(x2, y2)[0, 0]
    else:
        tc_total = _tc_loss(N - n_sc, N, V, 64)(x2, x2, y2, y2)[0, 0]
    eps = _SMOOTH / (V - 2)
    masked_total = tc_total + jnp.sum(ssum)
    return (jnp.sum(contrib) - eps * masked_total) / norm


# speedup vs baseline: 1.2429x; 1.0043x over previous
import functools
import math

import jax
import jax.numpy as jnp
from jax import lax
from jax.experimental import pallas as pl
from jax.experimental.pallas import tpu as pltpu
from jax.experimental.pallas import tpu_sc as plsc

_SMOOTH = 0.1


@functools.cache
def _sc_part(N, V, n_sc):
    info = plsc.get_sparse_core_info()
    nc, ns, nl = info.num_cores, info.num_subcores, info.num_lanes
    nw = nc * ns
    per_w = N // nw
    eps = _SMOOTH / (V - 2)
    cconst = eps * (V - 2) * math.log(eps) + (1.0 - _SMOOTH) * math.log(1.0 - _SMOOTH)
    mesh = plsc.VectorSubcoreMesh(core_axis_name="c", subcore_axis_name="s")

    cw = 1408
    vmain = (V // 128) * 128
    nch = vmain // cw
    assert nch * cw == vmain
    tpw = n_sc // 8 // nw
    assert tpw * 8 * nw == n_sc
    n_tc = N - n_sc
    y2len = max(nl, 8 * tpw)

    @functools.partial(
        pl.kernel,
        mesh=mesh,
        out_type=(
            jax.ShapeDtypeStruct((nw * nl,), jnp.float32),
            jax.ShapeDtypeStruct((nw * nl,), jnp.float32),
        ),
        scratch_types=[
            pltpu.SMEM((per_w,), jnp.int32),
            pltpu.VMEM((per_w,), jnp.int32),
            pltpu.VMEM((per_w, 8, 128), jnp.float32),
            pltpu.VMEM((per_w, 128), jnp.float32),
            pltpu.VMEM((nl,), jnp.float32),
            pltpu.SMEM((y2len,), jnp.int32),
            pltpu.VMEM((y2len,), jnp.int32),
            pltpu.VMEM((8, cw), jnp.float32),
            pltpu.VMEM((8, cw), jnp.float32),
            pltpu.VMEM((nl,), jnp.float32),
            pltpu.SemaphoreType.DMA,
            pltpu.SemaphoreType.DMA,
            pltpu.SemaphoreType.DMA,
        ],
    )
    def sc_kernel(
        x2d, yh, outh, out2h, y_s, y_vm, g_v, z_v, c_v, y2_s, y2_vm,
        sbuf, sbuf2, s_v, sem, semA, semB
    ):
        wid = lax.axis_index("s") * nc + lax.axis_index("c")
        base = wid * per_w
        pltpu.sync_copy(yh.at[pl.ds(base, per_w)], y_vm)
        for c in range(per_w // nl):
            yv16 = y_vm[pl.ds(c * nl, nl)]
            for j in range(nl):
                y_s[c * nl + j] = yv16[j]
        pltpu.sync_copy(x2d.at[pl.ds(base, per_w), pl.ds(0, 128)], z_v)
        for b in range(per_w // 8):
            descs = []
            for r8 in range(8):
                r = b * 8 + r8
                col = pl.multiple_of((y_s[r] // 128) * 128, 128)
                row0 = base + (r // 8) * 8
                descs.append(
                    pltpu.async_copy(
                        x2d.at[pl.ds(row0, 8), pl.ds(col, 128)], g_v.at[r], sem
                    )
                )
            for d in descs:
                d.wait()
        lanes = lax.iota(jnp.int32, nl)
        one = jnp.full((nl,), 1, jnp.int32)
        m0 = (one - jnp.minimum(lanes, one)).astype(jnp.float32)
        cvec0 = m0 * cconst
        acc = jnp.zeros((nl,), jnp.float32)
        for r in range(per_w):
            y_r = y_s[r]
            yv = jnp.full((nl,), y_r, jnp.int32)
            lc = pl.multiple_of(((y_r % 128) // nl) * nl, nl)
            gv = g_v[r, r % 8, pl.ds(lc, nl)]
            zv = z_v[r, pl.ds(0, nl)]
            gm = (one - jnp.minimum(jnp.abs(lanes - yv % nl), one)).astype(
                jnp.float32
            )
            pm = jnp.minimum(yv, one).astype(jnp.float32)
            row_vec = (
                gv * gm * (eps - (1.0 - _SMOOTH)) + zv * (m0 * eps) + cvec0
            )
            acc = acc + row_vec * pm
        c_v[...] = acc
        pltpu.sync_copy(c_v, outh.at[pl.ds(wid * nl, nl)])

        if tpw > 0:
            srow_base = n_tc + wid * 8 * tpw
            pltpu.sync_copy(
                yh.at[pl.ds(srow_base, 8 * tpw)], y2_vm.at[pl.ds(0, 8 * tpw)]
            )
            for c in range((8 * tpw + nl - 1) // nl):
                yv16 = y2_vm[pl.ds(c * nl, nl)]
                for j in range(nl):
                    if c * nl + j < 8 * tpw:
                        y2_s[c * nl + j] = yv16[j]
            sacc = jnp.zeros((nl,), jnp.float32)

            def _row_sums(buf, acc_c, pms):
                for r in range(8):

                    def inner(i, ps, r=r):
                        a, b = ps
                        for u in range(4):
                            offa = pl.multiple_of(i * 128 + u * 32, nl)
                            a = a + buf[r, pl.ds(offa, nl)]
                            b = b + buf[r, pl.ds(offa + nl, nl)]
                        return (a, b)

                    pa, pb = lax.fori_loop(
                        0, cw // 128,
                        inner,
                        (jnp.zeros((nl,), jnp.float32), jnp.zeros((nl,), jnp.float32)),
                    )
                    acc_c = acc_c + (pa + pb) * pms[r]
                return acc_c

            for t in range(tpw):
                row0 = pl.multiple_of(srow_base + t * 8, 8)
                pms = []
                for r in range(8):
                    yv = jnp.full((nl,), y2_s[t * 8 + r], jnp.int32)
                    pms.append(jnp.minimum(yv, one).astype(jnp.float32))

                def _issue(c, buf, s):
                    col = pl.multiple_of(c * cw, 128)
                    pltpu.async_copy(
                        x2d.at[pl.ds(row0, 8), pl.ds(col, cw)], buf, s
                    )

                def _wait(buf, s):
                    pltpu.make_async_copy(
                        x2d.at[pl.ds(row0, 8), pl.ds(0, cw)], buf, s
                    ).wait()

                _issue(0, sbuf, semA)
                _issue(1, sbuf2, semB)

                def pair_body(i, acc_c, pms=pms):
                    c = i * 2
                    _wait(sbuf, semA)
                    acc_c = _row_sums(sbuf, acc_c, pms)
                    _issue(c + 2, sbuf, semA)
                    _wait(sbuf2, semB)
                    acc_c = _row_sums(sbuf2, acc_c, pms)

                    @pl.when(c + 3 < nch)
                    def _():
                        _issue(c + 3, sbuf2, semB)

                    return acc_c

                sacc = lax.fori_loop(0, (nch - 1) // 2, pair_body, sacc)
                _wait(sbuf, semA)
                sacc = _row_sums(sbuf, sacc, pms)
            s_v[...] = sacc
            pltpu.sync_copy(s_v, out2h.at[pl.ds(wid * nl, nl)])
        else:
            s_v[...] = jnp.zeros((nl,), jnp.float32)
            pltpu.sync_copy(s_v, out2h.at[pl.ds(wid * nl, nl)])

    return sc_kernel


@functools.cache
def _tc_loss(n_tc, n_all, V, br):
    nblk = n_tc // br
    vmain = (V // 128) * 128
    vtail = V - vmain

    if n_tc == 0:
        def tail_body(xt_ref, yf_ref, out_ref):
            tmask = lax.broadcasted_iota(jnp.int32, (n_all, 128), 1) < vtail
            ts = jnp.sum(
                jnp.where(tmask, xt_ref[...], 0.0), axis=1, keepdims=True
            )
            ts = jnp.where(yf_ref[...] != 0, ts, 0.0)
            out_ref[0, 0] = jnp.sum(ts)

        return pl.pallas_call(
            tail_body,
            grid=(1,),
            in_specs=[
                pl.BlockSpec((n_all, 128), lambda i: (0, vmain // 128)),
                pl.BlockSpec((n_all, 1), lambda i: (0, 0)),
            ],
            out_specs=pl.BlockSpec(
                (1, 1), lambda i: (0, 0), memory_space=pltpu.SMEM
            ),
            out_shape=jax.ShapeDtypeStruct((1, 1), jnp.float32),
        )

    def body(xm_ref, xt_ref, y_ref, yf_ref, out_ref, acc_ref):
        pid = pl.program_id(0)

        @pl.when(pid == 0)
        def _():
            tmask = lax.broadcasted_iota(jnp.int32, (n_all, 128), 1) < vtail
            ts = jnp.sum(
                jnp.where(tmask, xt_ref[...], 0.0), axis=1, keepdims=True
            )
            ts = jnp.where(yf_ref[...] != 0, ts, 0.0)
            acc_ref[0] = jnp.sum(ts)

        srow = jnp.sum(xm_ref[...], axis=1, keepdims=True)
        srow = jnp.where(y_ref[...] != 0, srow, 0.0)
        acc_ref[0] += jnp.sum(srow)

        @pl.when(pid == nblk - 1)
        def _():
            out_ref[0, 0] = acc_ref[0]

    return pl.pallas_call(
        body,
        grid=(nblk,),
        in_specs=[
            pl.BlockSpec((br, vmain), lambda i: (i, 0)),
            pl.BlockSpec((n_all, 128), lambda i: (0, vmain // 128)),
            pl.BlockSpec((br, 1), lambda i: (i, 0)),
            pl.BlockSpec((n_all, 1), lambda i: (0, 0)),
        ],
        out_specs=pl.BlockSpec((1, 1), lambda i: (0, 0), memory_space=pltpu.SMEM),
        out_shape=jax.ShapeDtypeStruct((1, 1), jnp.float32),
        scratch_shapes=[pltpu.SMEM((1,), jnp.float32)],
    )


def kernel(x, y, norm):
    V = x.shape[-1]
    x2 = x.reshape(-1, V)
    N = x2.shape[0]
    yf = y.reshape(-1).astype(jnp.int32)
    n_sc = 0
    contrib, ssum = _sc_part(N, V, n_sc)(x2, yf)
    y2 = yf.reshape(N, 1)
    if n_sc == N:
        tc_total = _tc_loss(0, N, V, 64)(x2, y2)[0, 0]
    else:
        tc_total = _tc_loss(N - n_sc, N, V, 64)(x2, x2, y2, y2)[0, 0]
    eps = _SMOOTH / (V - 2)
    masked_total = tc_total + jnp.sum(ssum)
    return (jnp.sum(contrib) - eps * masked_total) / norm
